# Initial kernel scaffold; baseline (speedup 1.0000x reference)
import jax
import jax.numpy as jnp
from jax.experimental import pallas as pl


def kernel(node_features, edge_src, edge_dst, edge_sh, edge_weight_cutoff, edge_scalars, Wq, fck_W1, fck_W2, fcv_W1, fcv_W2, Wdot):
    raise NotImplementedError("write your pallas kernel here")


# speedup vs baseline: 2.8899x; 2.8899x over previous
import functools

import numpy as np
import jax
import jax.numpy as jnp
from jax import lax
from jax.experimental import pallas as pl
from jax.experimental.pallas import tpu as pltpu
from jax.experimental.pallas import tpu_sc as plsc

N = 10000
E = 160000
D_IN = 128
D_Q = 32
D_K = 32
D_OUT = 64
NB = 10
H = 16

_QSCALE = 1.0 / (32.0 * np.sqrt(128.0))
_HSCALE = 1.0 / np.sqrt(10.0)
_KVNORM = 1.0 / (4.0 * np.sqrt(128.0))

BLK = 128
NBLK = E // BLK
NW = 32
SC_ITERS = (NBLK + NW - 1) // NW
N_PAD = 10240
NPS = N_PAD // 16
TB = 640


def _silu(x):
    return x / (1.0 + jnp.exp(-x))


def _q_body(f_ref, wq_ref, wd_ref, o_ref):
    q = jnp.dot(f_ref[...], wq_ref[...], preferred_element_type=jnp.float32)
    qd = jnp.dot(q, wd_ref[...], preferred_element_type=jnp.float32) * _QSCALE
    o_ref[...] = jnp.concatenate(
        [qd, jnp.zeros((N, D_IN - D_Q), jnp.float32)], axis=1)


def _gather_body(src_ref, dst_ref, f_ref, qd_ref, fs_out, qdd_out,
                 idx_v, frows_v, qrows_v, sem):
    wid = lax.axis_index("s") * 2 + lax.axis_index("c")

    def step(i, carry):
        blk = wid + NW * i

        @pl.when(blk < NBLK)
        def _():
            base = pl.multiple_of(blk * BLK, 8)
            pltpu.sync_copy(src_ref.at[pl.ds(base, BLK)], idx_v)
            pltpu.async_copy(f_ref.at[idx_v], frows_v, sem).wait()
            pltpu.sync_copy(frows_v, fs_out.at[pl.ds(base, BLK)])
            pltpu.sync_copy(dst_ref.at[pl.ds(base, BLK)], idx_v)
            pltpu.async_copy(qd_ref.at[idx_v], qrows_v, sem).wait()
            pltpu.sync_copy(qrows_v, qdd_out.at[pl.ds(base, BLK)])

        return carry

    lax.fori_loop(0, SC_ITERS, step, 0)


def _fold(p, width):
    size = 16 * width
    while size > width:
        half = size // 2
        p = p[:, :half] + p[:, half:size]
        size = half
    return p


def _edge_body(fs_ref, qdd_ref, sc_ref, sh_ref, cw_ref, wb_ref, w1k_ref,
               w1v_ref, rk_ref, rv_ref, o_ref):
    scal = sc_ref[...]
    hk = _silu(jnp.dot(scal, w1k_ref[...], preferred_element_type=jnp.float32) * _HSCALE)
    hv = _silu(jnp.dot(scal, w1v_ref[...], preferred_element_type=jnp.float32) * _HSCALE)
    hkx = jnp.dot(hk, rk_ref[...], preferred_element_type=jnp.float32)
    hvx = jnp.dot(hv, rv_ref[...], preferred_element_type=jnp.float32)
    g = jnp.dot(fs_ref[...].astype(jnp.bfloat16), wb_ref[...],
                preferred_element_type=jnp.float32)
    k = _fold(g[:, :H * D_K] * hkx, D_K)
    v = _fold(g[:, H * D_K:] * hvx, D_OUT)
    sh0 = sh_ref[...] * _KVNORM
    dot = jnp.sum(qdd_ref[:, :D_Q] * k, axis=1, keepdims=True) * sh0
    w = jnp.sqrt(cw_ref[...]) * jnp.exp(0.5 * dot)
    o_ref[...] = jnp.concatenate(
        [v * (sh0 * w), jnp.zeros((TB, D_IN - D_OUT), jnp.float32)], axis=1)


def _scatter_body(dst_ref, sv_ref, zeros_ref, out_ref, acc_sh, idx_v, rows_v):
    c = lax.axis_index("c")
    s = lax.axis_index("s")
    wid = s * 2 + c
    pltpu.sync_copy(zeros_ref.at[pl.ds(s * NPS, NPS)], acc_sh.at[pl.ds(s * NPS, NPS)])
    plsc.subcore_barrier()

    def step(i, carry):
        blk = wid + NW * i

        @pl.when(blk < NBLK)
        def _():
            base = pl.multiple_of(blk * BLK, 8)
            pltpu.sync_copy(dst_ref.at[pl.ds(base, BLK)], idx_v)
            pltpu.sync_copy(sv_ref.at[pl.ds(base, BLK)], rows_v)
            pltpu.sync_copy(rows_v, acc_sh.at[idx_v], add=True)

        return carry

    lax.fori_loop(0, SC_ITERS, step, 0)
    plsc.subcore_barrier()
    pltpu.sync_copy(acc_sh.at[pl.ds(s * NPS, NPS)],
                    out_ref.at[pl.ds(c * N_PAD + s * NPS, NPS)])


def _norm_body(p_ref, o_ref):
    srow = (p_ref[:N, :] + p_ref[N_PAD:N_PAD + N, :])[:, :D_OUT]
    nrm = jnp.sqrt(jnp.sum(srow * srow, axis=1, keepdims=True))
    o_ref[...] = srow / jnp.maximum(nrm, 1e-12)


def kernel(node_features, edge_src, edge_dst, edge_sh, edge_weight_cutoff,
           edge_scalars, Wq, fck_W1, fck_W2, fcv_W1, fcv_W2, Wdot):
    f = node_features
    src = edge_src.astype(jnp.int32)
    dst = edge_dst.astype(jnp.int32)
    sh0 = edge_sh[:, 0:1]
    cw = edge_weight_cutoff[:, None]
    wbk = fck_W2.transpose(1, 0, 2).reshape(D_IN, H * D_K)
    wbv = fcv_W2.transpose(1, 0, 2).reshape(D_IN, H * D_OUT)
    wbig = jnp.concatenate([wbk, wbv], axis=1).astype(jnp.bfloat16)
    rk = jnp.asarray(np.kron(np.eye(H, dtype=np.float32), np.ones((1, D_K), np.float32)))
    rv = jnp.asarray(np.kron(np.eye(H, dtype=np.float32), np.ones((1, D_OUT), np.float32)))

    qd = pl.pallas_call(
        _q_body,
        out_shape=jax.ShapeDtypeStruct((N, D_IN), jnp.float32),
    )(f, Wq, Wdot)

    mesh = plsc.VectorSubcoreMesh(core_axis_name="c", subcore_axis_name="s")
    gather = pl.kernel(
        _gather_body,
        mesh=mesh,
        out_type=[jax.ShapeDtypeStruct((E, D_IN), jnp.float32),
                  jax.ShapeDtypeStruct((E, D_IN), jnp.float32)],
        scratch_types=[pltpu.VMEM((BLK,), jnp.int32),
                       pltpu.VMEM((BLK, D_IN), jnp.float32),
                       pltpu.VMEM((BLK, D_IN), jnp.float32),
                       pltpu.SemaphoreType.DMA],
    )
    f_src, qdd = gather(src, dst, f, qd)

    sv = pl.pallas_call(
        _edge_body,
        grid=(E // TB,),
        in_specs=[
            pl.BlockSpec((TB, D_IN), lambda i: (i, 0)),
            pl.BlockSpec((TB, D_IN), lambda i: (i, 0)),
            pl.BlockSpec((TB, NB), lambda i: (i, 0)),
            pl.BlockSpec((TB, 1), lambda i: (i, 0)),
            pl.BlockSpec((TB, 1), lambda i: (i, 0)),
            pl.BlockSpec((D_IN, H * (D_K + D_OUT)), lambda i: (0, 0)),
            pl.BlockSpec((NB, H), lambda i: (0, 0)),
            pl.BlockSpec((NB, H), lambda i: (0, 0)),
            pl.BlockSpec((H, H * D_K), lambda i: (0, 0)),
            pl.BlockSpec((H, H * D_OUT), lambda i: (0, 0)),
        ],
        out_specs=pl.BlockSpec((TB, D_IN), lambda i: (i, 0)),
        out_shape=jax.ShapeDtypeStruct((E, D_IN), jnp.float32),
    )(f_src, qdd, edge_scalars, sh0, cw, wbig, fck_W1, fcv_W1, rk, rv)

    zeros = jnp.zeros((N_PAD, D_IN), jnp.float32)
    scat = pl.kernel(
        _scatter_body,
        mesh=mesh,
        out_type=jax.ShapeDtypeStruct((2 * N_PAD, D_IN), jnp.float32),
        scratch_types=[pltpu.VMEM_SHARED((N_PAD, D_IN), jnp.float32),
                       pltpu.VMEM((BLK,), jnp.int32),
                       pltpu.VMEM((BLK, D_IN), jnp.float32)],
    )
    parts = scat(dst, sv, zeros)

    out = pl.pallas_call(
        _norm_body,
        out_shape=jax.ShapeDtypeStruct((N, D_OUT), jnp.float32),
    )(parts)
    return out


# speedup vs baseline: 3.2624x; 1.1289x over previous
import functools

import numpy as np
import jax
import jax.numpy as jnp
from jax import lax
from jax.experimental import pallas as pl
from jax.experimental.pallas import tpu as pltpu
from jax.experimental.pallas import tpu_sc as plsc

N = 10000
E = 160000
D_IN = 128
D_Q = 32
D_K = 32
D_OUT = 64
NB = 10
H = 16

_QSCALE = 1.0 / (32.0 * np.sqrt(128.0))
_HSCALE = 1.0 / np.sqrt(10.0)
_KVNORM = 1.0 / (4.0 * np.sqrt(128.0))

BLK = 128
NBLK = E // BLK
NW = 32
SC_ITERS = (NBLK + NW - 1) // NW
N_PAD = 10240
NPS = N_PAD // 16
TB = 1600


def _silu(x):
    return x / (1.0 + jnp.exp(-x))


def _q_body(f_ref, wq_ref, wd_ref, o_ref):
    q = jnp.dot(f_ref[...], wq_ref[...], preferred_element_type=jnp.float32)
    qd = jnp.dot(q, wd_ref[...], preferred_element_type=jnp.float32) * _QSCALE
    o_ref[...] = jnp.concatenate(
        [qd, jnp.zeros((N, D_IN - D_Q), jnp.float32)], axis=1)


def _gather_body(src_ref, dst_ref, f_ref, qd_ref, fs_out, qdd_out,
                 idx_v, frows_v, qrows_v, sem):
    wid = lax.axis_index("s") * 2 + lax.axis_index("c")

    def step(i, carry):
        blk = wid + NW * i

        @pl.when(blk < NBLK)
        def _():
            base = pl.multiple_of(blk * BLK, 8)
            pltpu.sync_copy(src_ref.at[pl.ds(base, BLK)], idx_v)
            pltpu.async_copy(f_ref.at[idx_v], frows_v, sem).wait()
            pltpu.sync_copy(frows_v, fs_out.at[pl.ds(base, BLK)])
            pltpu.sync_copy(dst_ref.at[pl.ds(base, BLK)], idx_v)
            pltpu.async_copy(qd_ref.at[idx_v], qrows_v, sem).wait()
            pltpu.sync_copy(qrows_v, qdd_out.at[pl.ds(base, BLK)])

        return carry

    lax.fori_loop(0, SC_ITERS, step, 0)


def _edge_body(fs_ref, qdd_ref, sc_ref, sh_ref, cw_ref, wb_ref, w1k_ref,
               w1v_ref, rk_ref, rv_ref, sk_ref, sv_ref, ones_ref, o_ref):
    scal = sc_ref[...]
    hk = _silu(jnp.dot(scal, w1k_ref[...], preferred_element_type=jnp.float32) * _HSCALE)
    hv = _silu(jnp.dot(scal, w1v_ref[...], preferred_element_type=jnp.float32) * _HSCALE)
    hkx = jnp.dot(hk.astype(jnp.bfloat16), rk_ref[...],
                  preferred_element_type=jnp.float32)
    hvx = jnp.dot(hv.astype(jnp.bfloat16), rv_ref[...],
                  preferred_element_type=jnp.float32)
    g = jnp.dot(fs_ref[...].astype(jnp.bfloat16), wb_ref[...],
                preferred_element_type=jnp.float32)
    k = jnp.dot((g[:, :H * D_K] * hkx).astype(jnp.bfloat16), sk_ref[...],
                preferred_element_type=jnp.float32)
    v = jnp.dot((g[:, H * D_K:] * hvx).astype(jnp.bfloat16), sv_ref[...],
                preferred_element_type=jnp.float32)
    sh0 = sh_ref[...] * _KVNORM
    kq = qdd_ref[:, :D_Q] * k
    dot = jnp.dot(kq, ones_ref[...], preferred_element_type=jnp.float32) * sh0
    w = jnp.sqrt(cw_ref[...]) * jnp.exp(0.5 * dot)
    o_ref[...] = jnp.concatenate(
        [v * (sh0 * w), jnp.zeros((TB, D_IN - D_OUT), jnp.float32)], axis=1)


def _scatter_body(dst_ref, sv_ref, zeros_ref, out_ref, acc_sh, idx_v, rows_v):
    c = lax.axis_index("c")
    s = lax.axis_index("s")
    wid = s * 2 + c
    pltpu.sync_copy(zeros_ref.at[pl.ds(s * NPS, NPS)], acc_sh.at[pl.ds(s * NPS, NPS)])
    plsc.subcore_barrier()

    def step(i, carry):
        blk = wid + NW * i

        @pl.when(blk < NBLK)
        def _():
            base = pl.multiple_of(blk * BLK, 8)
            pltpu.sync_copy(dst_ref.at[pl.ds(base, BLK)], idx_v)
            pltpu.sync_copy(sv_ref.at[pl.ds(base, BLK)], rows_v)
            pltpu.sync_copy(rows_v, acc_sh.at[idx_v], add=True)

        return carry

    lax.fori_loop(0, SC_ITERS, step, 0)
    plsc.subcore_barrier()
    pltpu.sync_copy(acc_sh.at[pl.ds(s * NPS, NPS)],
                    out_ref.at[pl.ds(c * N_PAD + s * NPS, NPS)])


def _norm_body(p_ref, o_ref):
    srow = (p_ref[:N, :] + p_ref[N_PAD:N_PAD + N, :])[:, :D_OUT]
    nrm = jnp.sqrt(jnp.sum(srow * srow, axis=1, keepdims=True))
    o_ref[...] = srow / jnp.maximum(nrm, 1e-12)


def kernel(node_features, edge_src, edge_dst, edge_sh, edge_weight_cutoff,
           edge_scalars, Wq, fck_W1, fck_W2, fcv_W1, fcv_W2, Wdot):
    f = node_features
    src = edge_src.astype(jnp.int32)
    dst = edge_dst.astype(jnp.int32)
    sh0 = edge_sh[:, 0:1]
    cw = edge_weight_cutoff[:, None]
    wbk = fck_W2.transpose(1, 0, 2).reshape(D_IN, H * D_K)
    wbv = fcv_W2.transpose(1, 0, 2).reshape(D_IN, H * D_OUT)
    wbig = jnp.concatenate([wbk, wbv], axis=1).astype(jnp.bfloat16)
    rk = jnp.asarray(np.kron(np.eye(H, dtype=np.float32), np.ones((1, D_K), np.float32)).astype(np.float32)).astype(jnp.bfloat16)
    rv = jnp.asarray(np.kron(np.eye(H, dtype=np.float32), np.ones((1, D_OUT), np.float32))).astype(jnp.bfloat16)
    sk = jnp.asarray(np.kron(np.ones((H, 1), np.float32), np.eye(D_K, dtype=np.float32))).astype(jnp.bfloat16)
    svm = jnp.asarray(np.kron(np.ones((H, 1), np.float32), np.eye(D_OUT, dtype=np.float32))).astype(jnp.bfloat16)
    ones_col = jnp.ones((D_Q, 1), jnp.float32)

    qd = pl.pallas_call(
        _q_body,
        out_shape=jax.ShapeDtypeStruct((N, D_IN), jnp.float32),
    )(f, Wq, Wdot)

    mesh = plsc.VectorSubcoreMesh(core_axis_name="c", subcore_axis_name="s")
    gather = pl.kernel(
        _gather_body,
        mesh=mesh,
        out_type=[jax.ShapeDtypeStruct((E, D_IN), jnp.float32),
                  jax.ShapeDtypeStruct((E, D_IN), jnp.float32)],
        scratch_types=[pltpu.VMEM((BLK,), jnp.int32),
                       pltpu.VMEM((BLK, D_IN), jnp.float32),
                       pltpu.VMEM((BLK, D_IN), jnp.float32),
                       pltpu.SemaphoreType.DMA],
    )
    f_src, qdd = gather(src, dst, f, qd)

    sv = pl.pallas_call(
        _edge_body,
        grid=(E // TB,),
        in_specs=[
            pl.BlockSpec((TB, D_IN), lambda i: (i, 0)),
            pl.BlockSpec((TB, D_IN), lambda i: (i, 0)),
            pl.BlockSpec((TB, NB), lambda i: (i, 0)),
            pl.BlockSpec((TB, 1), lambda i: (i, 0)),
            pl.BlockSpec((TB, 1), lambda i: (i, 0)),
            pl.BlockSpec((D_IN, H * (D_K + D_OUT)), lambda i: (0, 0)),
            pl.BlockSpec((NB, H), lambda i: (0, 0)),
            pl.BlockSpec((NB, H), lambda i: (0, 0)),
            pl.BlockSpec((H, H * D_K), lambda i: (0, 0)),
            pl.BlockSpec((H, H * D_OUT), lambda i: (0, 0)),
            pl.BlockSpec((H * D_K, D_K), lambda i: (0, 0)),
            pl.BlockSpec((H * D_OUT, D_OUT), lambda i: (0, 0)),
            pl.BlockSpec((D_Q, 1), lambda i: (0, 0)),
        ],
        out_specs=pl.BlockSpec((TB, D_IN), lambda i: (i, 0)),
        out_shape=jax.ShapeDtypeStruct((E, D_IN), jnp.float32),
    )(f_src, qdd, edge_scalars, sh0, cw, wbig, fck_W1, fcv_W1, rk, rv, sk, svm, ones_col)

    zeros = jnp.zeros((N_PAD, D_IN), jnp.float32)
    scat = pl.kernel(
        _scatter_body,
        mesh=mesh,
        out_type=jax.ShapeDtypeStruct((2 * N_PAD, D_IN), jnp.float32),
        scratch_types=[pltpu.VMEM_SHARED((N_PAD, D_IN), jnp.float32),
                       pltpu.VMEM((BLK,), jnp.int32),
                       pltpu.VMEM((BLK, D_IN), jnp.float32)],
    )
    parts = scat(dst, sv, zeros)

    out = pl.pallas_call(
        _norm_body,
        out_shape=jax.ShapeDtypeStruct((N, D_OUT), jnp.float32),
    )(parts)
    return out


# speedup vs baseline: 4.3875x; 1.3449x over previous
import functools

import numpy as np
import jax
import jax.numpy as jnp
from jax import lax
from jax.experimental import pallas as pl
from jax.experimental.pallas import tpu as pltpu
from jax.experimental.pallas import tpu_sc as plsc

N = 10000
E = 160000
D_IN = 128
D_Q = 32
D_K = 32
D_OUT = 64
NB = 10
H = 16

_QSCALE = 1.0 / (32.0 * np.sqrt(128.0))
_HSCALE = 1.0 / np.sqrt(10.0)
_KVNORM = 1.0 / (4.0 * np.sqrt(128.0))

BLK = 128
NBLK = E // BLK
NW = 32
SC_ITERS = (NBLK + NW - 1) // NW
N_PAD = 10240
NPS = N_PAD // 16
TB = 3200


def _silu(x):
    return x / (1.0 + jnp.exp(-x))


def _q_body(f_ref, wq_ref, wd_ref, o_ref):
    q = jnp.dot(f_ref[...], wq_ref[...], preferred_element_type=jnp.float32)
    qd = jnp.dot(q, wd_ref[...], preferred_element_type=jnp.float32) * _QSCALE
    o_ref[...] = jnp.concatenate(
        [qd, jnp.zeros((N, D_IN - D_Q), jnp.float32)], axis=1)


def _make_gather_body(nblk, iters):
  def _gather_body(src_ref, dst_ref, f_ref, qd_ref, fs_out, qdd_out,
                 idxf, idxq, frows, qrows,
                 sif0, sif1, siq0, siq1, sgf0, sgf1, sgq0, sgq1,
                 sof0, sof1, soq0, soq1):
    wid = lax.axis_index("s") * 2 + lax.axis_index("c")
    sif = (sif0, sif1); siq = (siq0, siq1)
    sgf = (sgf0, sgf1); sgq = (sgq0, sgq1)
    sof = (sof0, sof1); soq = (soq0, soq1)

    def blk_of(g):
        return wid + NW * g

    def issue_idx(g, b):
        @pl.when(blk_of(g) < nblk)
        def _():
            base = pl.multiple_of(blk_of(g) * BLK, 8)
            pltpu.async_copy(src_ref.at[pl.ds(base, BLK)], idxf.at[b], sif[b])
            pltpu.async_copy(dst_ref.at[pl.ds(base, BLK)], idxq.at[b], siq[b])

    issue_idx(0, 0)

    def half(g, b):
        valid = blk_of(g) < nblk

        @pl.when(valid)
        def _():
            pltpu.make_async_copy(src_ref.at[pl.ds(0, BLK)], idxf.at[b], sif[b]).wait()
            pltpu.make_async_copy(dst_ref.at[pl.ds(0, BLK)], idxq.at[b], siq[b]).wait()

        @pl.when((g >= 2) & (blk_of(g - 2) < nblk))
        def _():
            pltpu.make_async_copy(frows.at[b], fs_out.at[pl.ds(0, BLK)], sof[b]).wait()
            pltpu.make_async_copy(qrows.at[b], qdd_out.at[pl.ds(0, BLK)], soq[b]).wait()

        @pl.when(valid)
        def _():
            pltpu.async_copy(f_ref.at[idxf.at[b]], frows.at[b], sgf[b])
            pltpu.async_copy(qd_ref.at[idxq.at[b]], qrows.at[b], sgq[b])

        @pl.when((g >= 1) & (blk_of(g - 1) < nblk))
        def _():
            nb = 1 - b
            base = pl.multiple_of(blk_of(g - 1) * BLK, 8)
            pltpu.make_async_copy(f_ref.at[idxf.at[nb]], frows.at[nb], sgf[nb]).wait()
            pltpu.make_async_copy(qd_ref.at[idxq.at[nb]], qrows.at[nb], sgq[nb]).wait()
            pltpu.async_copy(frows.at[nb], fs_out.at[pl.ds(base, BLK)], sof[nb])
            pltpu.async_copy(qrows.at[nb], qdd_out.at[pl.ds(base, BLK)], soq[nb])

        issue_idx(g + 1, 1 - b)

    def step(g2, carry):
        half(2 * g2, 0)
        half(2 * g2 + 1, 1)
        return carry

    lax.fori_loop(0, iters // 2, step, 0)
    gl = iters - 1
    bl = gl % 2

    @pl.when(blk_of(gl) < nblk)
    def _():
        base = pl.multiple_of(blk_of(gl) * BLK, 8)
        pltpu.make_async_copy(f_ref.at[idxf.at[bl]], frows.at[bl], sgf[bl]).wait()
        pltpu.make_async_copy(qd_ref.at[idxq.at[bl]], qrows.at[bl], sgq[bl]).wait()
        pltpu.async_copy(frows.at[bl], fs_out.at[pl.ds(base, BLK)], sof[bl])
        pltpu.async_copy(qrows.at[bl], qdd_out.at[pl.ds(base, BLK)], soq[bl])

    for g in (iters - 2, iters - 1):
        b = g % 2

        @pl.when(blk_of(g) < nblk)
        def _():
            pltpu.make_async_copy(frows.at[b], fs_out.at[pl.ds(0, BLK)], sof[b]).wait()
            pltpu.make_async_copy(qrows.at[b], qdd_out.at[pl.ds(0, BLK)], soq[b]).wait()

  return _gather_body


def _edge_body(fs_ref, qdd_ref, sc_ref, sh_ref, cw_ref, wb_ref, w1a_ref,
               ra_ref, tq_ref, sv_ref, ones_ref, o_ref):
    scal = sc_ref[...]
    ha = _silu(jnp.dot(scal, w1a_ref[...], preferred_element_type=jnp.float32) * _HSCALE)
    hx = jnp.dot(ha.astype(jnp.bfloat16), ra_ref[...],
                 preferred_element_type=jnp.float32).astype(jnp.bfloat16)
    hkx = hx[:, :H * D_K]
    hvx = hx[:, H * D_K:]
    g = jnp.dot(fs_ref[...].astype(jnp.bfloat16), wb_ref[...],
                preferred_element_type=jnp.float32).astype(jnp.bfloat16)

    def _fold(p, width):
        size = p.shape[1]
        while size > width:
            half = size // 2
            p = p[:, :half] + p[:, half:size]
            size = half
        return p

    k = _fold(g[:, :H * D_K] * hkx, D_K).astype(jnp.float32)
    v = _fold(g[:, H * D_K:] * hvx, D_OUT).astype(jnp.float32)
    sh0 = sh_ref[...] * _KVNORM
    kq = qdd_ref[:, :D_Q] * k
    dot = jnp.dot(kq, ones_ref[...], preferred_element_type=jnp.float32) * sh0
    w = jnp.sqrt(cw_ref[...]) * jnp.exp(0.5 * dot)
    o_ref[...] = jnp.concatenate(
        [v * (sh0 * w), jnp.zeros((TB, D_IN - D_OUT), jnp.float32)], axis=1)


def _make_scatter_body(nblk, iters):
  def _scatter_body(dst_ref, sv_ref, zeros_ref, out_ref, acc_sh, idx_v, rows_v,
                  si0, si1, sr0, sr1, sc0, sc1):
    c = lax.axis_index("c")
    s = lax.axis_index("s")
    wid = s * 2 + c
    si = (si0, si1); sr = (sr0, sr1); sca = (sc0, sc1)
    pltpu.sync_copy(zeros_ref.at[pl.ds(s * NPS, NPS)], acc_sh.at[pl.ds(s * NPS, NPS)])
    plsc.subcore_barrier()

    def blk_of(g):
        return wid + NW * g

    def issue_loads(g, b):
        @pl.when(blk_of(g) < nblk)
        def _():
            base = pl.multiple_of(blk_of(g) * BLK, 8)
            pltpu.async_copy(dst_ref.at[pl.ds(base, BLK)], idx_v.at[b], si[b])
            pltpu.async_copy(sv_ref.at[pl.ds(base, BLK)], rows_v.at[b], sr[b])

    issue_loads(0, 0)

    def half(g, b):
        valid = blk_of(g) < nblk

        @pl.when(valid)
        def _():
            pltpu.make_async_copy(dst_ref.at[pl.ds(0, BLK)], idx_v.at[b], si[b]).wait()
            pltpu.make_async_copy(sv_ref.at[pl.ds(0, BLK)], rows_v.at[b], sr[b]).wait()
            pltpu.async_copy(rows_v.at[b], acc_sh.at[idx_v.at[b]], sca[b], add=True)

        @pl.when((g >= 1) & (blk_of(g - 1) < nblk))
        def _():
            pltpu.make_async_copy(rows_v.at[1 - b], acc_sh.at[idx_v.at[1 - b]],
                                  sca[1 - b]).wait()

        issue_loads(g + 1, 1 - b)

    def step(g2, carry):
        half(2 * g2, 0)
        half(2 * g2 + 1, 1)
        return carry

    lax.fori_loop(0, iters // 2, step, 0)

    g_last = iters - 1
    @pl.when(blk_of(g_last) < nblk)
    def _():
        pltpu.make_async_copy(rows_v.at[g_last % 2], acc_sh.at[idx_v.at[g_last % 2]],
                              sca[g_last % 2]).wait()

    plsc.subcore_barrier()
    pltpu.sync_copy(acc_sh.at[pl.ds(s * NPS, NPS)],
                    out_ref.at[pl.ds(c * N_PAD + s * NPS, NPS)])

  return _scatter_body


def _norm_body(p0_ref, p1_ref, o_ref):
    srow = (p0_ref[:N, :] + p0_ref[N_PAD:N_PAD + N, :]
            + p1_ref[:N, :] + p1_ref[N_PAD:N_PAD + N, :])[:, :D_OUT]
    nrm = jnp.sqrt(jnp.sum(srow * srow, axis=1, keepdims=True))
    o_ref[...] = srow / jnp.maximum(nrm, 1e-12)


def kernel(node_features, edge_src, edge_dst, edge_sh, edge_weight_cutoff,
           edge_scalars, Wq, fck_W1, fck_W2, fcv_W1, fcv_W2, Wdot):
    f = node_features
    src = edge_src.astype(jnp.int32)
    dst = edge_dst.astype(jnp.int32)
    sh0 = edge_sh[:, 0:1]
    cw = edge_weight_cutoff[:, None]
    wbk = fck_W2.transpose(1, 0, 2).reshape(D_IN, H * D_K)
    wbv = fcv_W2.transpose(1, 0, 2).reshape(D_IN, H * D_OUT)
    wbig = jnp.concatenate([wbk, wbv], axis=1).astype(jnp.bfloat16)
    rk = np.kron(np.eye(H, dtype=np.float32), np.ones((1, D_K), np.float32))
    rv = np.kron(np.eye(H, dtype=np.float32), np.ones((1, D_OUT), np.float32))
    ra = np.zeros((2 * H, H * (D_K + D_OUT)), np.float32)
    ra[:H, :H * D_K] = rk
    ra[H:, H * D_K:] = rv
    ra = jnp.asarray(ra).astype(jnp.bfloat16)
    w1a = jnp.concatenate([fck_W1, fcv_W1], axis=1)
    tq = jnp.asarray(np.kron(np.ones((H, 1), np.float32), np.eye(D_K, dtype=np.float32))).astype(jnp.bfloat16)
    svm = jnp.asarray(np.kron(np.ones((H, 1), np.float32), np.eye(D_OUT, dtype=np.float32))).astype(jnp.bfloat16)
    ones_col = jnp.ones((D_Q, 1), jnp.float32)

    qd = pl.pallas_call(
        _q_body,
        out_shape=jax.ShapeDtypeStruct((N, D_IN), jnp.float32),
    )(f, Wq, Wdot)

    mesh = plsc.VectorSubcoreMesh(core_axis_name="c", subcore_axis_name="s")
    zeros = jnp.zeros((N_PAD, D_IN), jnp.float32)

    EC = E // 2
    NBLK_C = EC // BLK
    ITERS_C = (NBLK_C + NW - 1) // NW

    parts = []
    for ci in range(2):
        src_c = src[ci * EC:(ci + 1) * EC]
        dst_c = dst[ci * EC:(ci + 1) * EC]
        gather = pl.kernel(
            _make_gather_body(NBLK_C, ITERS_C),
            mesh=mesh,
            out_type=[jax.ShapeDtypeStruct((EC, D_IN), jnp.float32),
                      jax.ShapeDtypeStruct((EC, D_IN), jnp.float32)],
            scratch_types=[pltpu.VMEM((2, BLK), jnp.int32),
                           pltpu.VMEM((2, BLK), jnp.int32),
                           pltpu.VMEM((2, BLK, D_IN), jnp.float32),
                           pltpu.VMEM((2, BLK, D_IN), jnp.float32)]
                          + [pltpu.SemaphoreType.DMA] * 12,
        )
        f_src, qdd = gather(src_c, dst_c, f, qd)

        goff = ci * (EC // TB)
        sv = pl.pallas_call(
            _edge_body,
            grid=(EC // TB,),
            in_specs=[
                pl.BlockSpec((TB, D_IN), lambda i: (i, 0)),
                pl.BlockSpec((TB, D_IN), lambda i: (i, 0)),
                pl.BlockSpec((TB, NB), lambda i, o=goff: (i + o, 0)),
                pl.BlockSpec((TB, 1), lambda i, o=goff: (i + o, 0)),
                pl.BlockSpec((TB, 1), lambda i, o=goff: (i + o, 0)),
                pl.BlockSpec((D_IN, H * (D_K + D_OUT)), lambda i: (0, 0)),
                pl.BlockSpec((NB, 2 * H), lambda i: (0, 0)),
                pl.BlockSpec((2 * H, H * (D_K + D_OUT)), lambda i: (0, 0)),
                pl.BlockSpec((H * D_K, D_K), lambda i: (0, 0)),
                pl.BlockSpec((H * D_OUT, D_OUT), lambda i: (0, 0)),
                pl.BlockSpec((D_Q, 1), lambda i: (0, 0)),
            ],
            out_specs=pl.BlockSpec((TB, D_IN), lambda i: (i, 0)),
            out_shape=jax.ShapeDtypeStruct((EC, D_IN), jnp.float32),
        )(f_src, qdd, edge_scalars, sh0, cw, wbig, w1a, ra, tq,
          svm, ones_col)

        scat = pl.kernel(
            _make_scatter_body(NBLK_C, ITERS_C),
            mesh=mesh,
            out_type=jax.ShapeDtypeStruct((2 * N_PAD, D_IN), jnp.float32),
            scratch_types=[pltpu.VMEM_SHARED((N_PAD, D_IN), jnp.float32),
                           pltpu.VMEM((2, BLK), jnp.int32),
                           pltpu.VMEM((2, BLK, D_IN), jnp.float32)]
                          + [pltpu.SemaphoreType.DMA] * 6,
        )
        parts.append(scat(dst_c, sv, zeros))

    out = pl.pallas_call(
        _norm_body,
        out_shape=jax.ShapeDtypeStruct((N, D_OUT), jnp.float32),
    )(parts[0], parts[1])
    return out
